# baseline (speedup 1.0000x reference)
import functools
import math

import jax
import jax.numpy as jnp
from jax import lax
from jax.experimental import pallas as pl
from jax.experimental.pallas import tpu as pltpu
from jax.experimental.pallas import tpu_sc as plsc

LOG2_T = math.log2(0.2)

N = 2048
B = 8
TR = 256
NR = N // TR


def _stage1_body(xa_ref, xb_ref, max0_ref, idx0_ref, max1_ref, idx1_ref):
    r = pl.program_id(0)

    @pl.when(r == 0)
    def _():
        max1_ref[...] = jnp.full((B, N), -jnp.inf, jnp.float32)
        idx1_ref[...] = jnp.zeros((B, N), jnp.int32)

    lane_iota = lax.broadcasted_iota(jnp.int32, (8, B, 128), 2)
    rmax_parts = []
    ridx_parts = []
    half_chunks = TR // 16
    for c in range(TR // 8):
        x_ref = xa_ref if c < half_chunks else xb_ref
        c_loc = c if c < half_chunks else c - half_chunks
        rows = slice(8 * c_loc, 8 * (c_loc + 1))
        racc = None
        for g in range(N // 128):
            cols = slice(128 * g, 128 * (g + 1))
            v = x_ref[rows, :, cols]
            if g == 0:
                racc = v
                racci = lane_iota
            else:
                gt = v > racc
                racc = jnp.where(gt, v, racc)
                racci = jnp.where(gt, lane_iota + g * 128, racci)
            cacc = max1_ref[:, cols]
            cacci = idx1_ref[:, cols]
            for t in range(8):
                vt = v[t]
                gt_c = vt > cacc
                cacc = jnp.where(gt_c, vt, cacc)
                cacci = jnp.where(gt_c, jnp.int32(r * TR + 8 * c + t), cacci)
            max1_ref[:, cols] = cacc
            idx1_ref[:, cols] = cacci
        rmax = jnp.max(racc, axis=2)
        ridx = jnp.min(jnp.where(racc == rmax[:, :, None], racci, N), axis=2)
        rmax_parts.append(rmax.T)
        ridx_parts.append(ridx.T)
    max0_ref[...] = jnp.concatenate(rmax_parts, axis=1)
    idx0_ref[...] = jnp.concatenate(ridx_parts, axis=1)


def _stage1(scores):
    scores_t = jnp.transpose(scores, (1, 0, 2))
    out_shapes = (
        jax.ShapeDtypeStruct((B, N), jnp.float32),
        jax.ShapeDtypeStruct((B, N), jnp.int32),
        jax.ShapeDtypeStruct((B, N), jnp.float32),
        jax.ShapeDtypeStruct((B, N), jnp.int32),
    )
    in_specs = [
        pl.BlockSpec((TR // 2, B, N), lambda r: (2 * r, 0, 0)),
        pl.BlockSpec((TR // 2, B, N), lambda r: (2 * r + 1, 0, 0)),
    ]
    out_specs = (
        pl.BlockSpec((B, TR), lambda r: (0, r)),
        pl.BlockSpec((B, TR), lambda r: (0, r)),
        pl.BlockSpec((B, N), lambda r: (0, 0)),
        pl.BlockSpec((B, N), lambda r: (0, 0)),
    )
    max0, idx0, max1, idx1 = pl.pallas_call(
        _stage1_body,
        grid=(NR,),
        in_specs=in_specs,
        out_specs=out_specs,
        out_shape=out_shapes,
        compiler_params=pltpu.CompilerParams(
            dimension_semantics=("arbitrary",),
        ),
    )(scores_t, scores_t)
    return max0, idx0, max1, idx1


_SC_INFO = None


def _sc_info():
    global _SC_INFO
    if _SC_INFO is None:
        info = plsc.get_sparse_core_info()
        _SC_INFO = (info.num_cores, info.num_subcores, info.num_lanes)
    return _SC_INFO


def _stage2(idx0, idx1, max0, max1):
    nc, ns, nl = _sc_info()
    nw = nc * ns
    chunks_per_b = nw // B
    chunk = N // chunks_per_b

    mesh = plsc.VectorSubcoreMesh(core_axis_name="c", subcore_axis_name="s")
    out_type = (
        jax.ShapeDtypeStruct((B, N), jnp.int32),
        jax.ShapeDtypeStruct((B, N), jnp.int32),
        jax.ShapeDtypeStruct((B, N), jnp.float32),
        jax.ShapeDtypeStruct((B, N), jnp.float32),
    )

    @functools.partial(
        pl.kernel,
        mesh=mesh,
        out_type=out_type,
        compiler_params=pltpu.CompilerParams(needs_layout_passes=False),
        scratch_types=[
            pltpu.VMEM((N,), jnp.int32),
            pltpu.VMEM((N,), jnp.int32),
            pltpu.VMEM((chunk,), jnp.float32),
            pltpu.VMEM((chunk,), jnp.float32),
            pltpu.VMEM((chunk,), jnp.int32),
            pltpu.VMEM((chunk,), jnp.int32),
            pltpu.VMEM((chunk,), jnp.float32),
            pltpu.VMEM((chunk,), jnp.float32),
            pltpu.SemaphoreType.DMA,
            pltpu.SemaphoreType.DMA,
            pltpu.SemaphoreType.DMA,
            pltpu.SemaphoreType.DMA,
        ],
    )
    def sc_kernel(idx0_hbm, idx1_hbm, max0_hbm, max1_hbm,
                  oi0_hbm, oi1_hbm, os0_hbm, os1_hbm,
                  idx0_v, idx1_v, max0_v, max1_v,
                  oi0_v, oi1_v, os0_v, os1_v,
                  sem0, sem1, sem2, sem3):
        wid = lax.axis_index("s") * nc + lax.axis_index("c")
        b = wid // chunks_per_b
        base = (wid % chunks_per_b) * chunk

        c0 = pltpu.async_copy(idx0_hbm.at[b], idx0_v, sem0)
        c1 = pltpu.async_copy(idx1_hbm.at[b], idx1_v, sem1)
        c2 = pltpu.async_copy(max0_hbm.at[b, pl.ds(base, chunk)], max0_v, sem2)
        c3 = pltpu.async_copy(max1_hbm.at[b, pl.ds(base, chunk)], max1_v, sem3)
        c0.wait()
        c1.wait()
        c2.wait()
        c3.wait()

        thr = jnp.float32(LOG2_T)
        for v in range(chunk // nl):
            s = v * nl
            ivec = lax.iota(jnp.int32, nl) + (base + s)
            a0 = idx0_v[pl.ds(base + s, nl)]
            g0 = plsc.load_gather(idx1_v, [a0])
            m0 = max0_v[pl.ds(s, nl)]
            valid0 = (g0 == ivec) & (m0 > thr)
            oi0_v[pl.ds(s, nl)] = jnp.where(valid0, a0, -1)
            os0_v[pl.ds(s, nl)] = jnp.where(valid0, m0, jnp.float32(0.0))
            a1 = idx1_v[pl.ds(base + s, nl)]
            g1 = plsc.load_gather(idx0_v, [a1])
            m1 = max1_v[pl.ds(s, nl)]
            valid1 = (g1 == ivec) & (m1 > thr)
            oi1_v[pl.ds(s, nl)] = jnp.where(valid1, a1, -1)
            os1_v[pl.ds(s, nl)] = jnp.where(valid1, m1, jnp.float32(0.0))

        o0 = pltpu.async_copy(oi0_v, oi0_hbm.at[b, pl.ds(base, chunk)], sem0)
        o1 = pltpu.async_copy(oi1_v, oi1_hbm.at[b, pl.ds(base, chunk)], sem1)
        o2 = pltpu.async_copy(os0_v, os0_hbm.at[b, pl.ds(base, chunk)], sem2)
        o3 = pltpu.async_copy(os1_v, os1_hbm.at[b, pl.ds(base, chunk)], sem3)
        o0.wait()
        o1.wait()
        o2.wait()
        o3.wait()

    return sc_kernel(idx0, idx1, max0, max1)


def kernel(scores):
    max0, idx0, max1, idx1 = _stage1(scores)
    return _stage2(idx0, idx1, max0, max1)


# speedup vs baseline: 1.0047x; 1.0047x over previous
import functools
import math

import jax
import jax.numpy as jnp
from jax import lax
from jax.experimental import pallas as pl
from jax.experimental.pallas import tpu as pltpu
from jax.experimental.pallas import tpu_sc as plsc

LOG2_T = math.log2(0.2)

N = 2048
B = 8
TR = 256
NR = N // TR


def _stage1_body(x_ref, max0_ref, idx0_ref, max1_ref, idx1_ref):
    r = pl.program_id(0)

    @pl.when(r == 0)
    def _():
        max1_ref[...] = jnp.full((B, N), -jnp.inf, jnp.float32)
        idx1_ref[...] = jnp.zeros((B, N), jnp.int32)

    lane_iota = lax.broadcasted_iota(jnp.int32, (8, B, 128), 2)
    rmax_parts = []
    ridx_parts = []
    for c in range(TR // 8):
        rows = slice(8 * c, 8 * (c + 1))
        racc = None
        for g in range(N // 128):
            cols = slice(128 * g, 128 * (g + 1))
            v = x_ref[rows, :, cols]
            if g == 0:
                racc = v
                racci = lane_iota
            else:
                gt = v > racc
                racc = jnp.where(gt, v, racc)
                racci = jnp.where(gt, lane_iota + g * 128, racci)
            cacc = max1_ref[:, cols]
            cacci = idx1_ref[:, cols]
            for t in range(8):
                vt = v[t]
                gt_c = vt > cacc
                cacc = jnp.where(gt_c, vt, cacc)
                cacci = jnp.where(gt_c, jnp.int32(r * TR + 8 * c + t), cacci)
            max1_ref[:, cols] = cacc
            idx1_ref[:, cols] = cacci
        rmax = jnp.max(racc, axis=2)
        ridx = jnp.min(jnp.where(racc == rmax[:, :, None], racci, N), axis=2)
        rmax_parts.append(rmax.T)
        ridx_parts.append(ridx.T)
    max0_ref[...] = jnp.concatenate(rmax_parts, axis=1)
    idx0_ref[...] = jnp.concatenate(ridx_parts, axis=1)


def _stage1(scores):
    scores_t = jnp.transpose(scores, (1, 0, 2))
    out_shapes = (
        jax.ShapeDtypeStruct((B, N), jnp.float32),
        jax.ShapeDtypeStruct((B, N), jnp.int32),
        jax.ShapeDtypeStruct((B, N), jnp.float32),
        jax.ShapeDtypeStruct((B, N), jnp.int32),
    )
    in_specs = [pl.BlockSpec((TR, B, N), lambda r: (r, 0, 0))]
    out_specs = (
        pl.BlockSpec((B, TR), lambda r: (0, r)),
        pl.BlockSpec((B, TR), lambda r: (0, r)),
        pl.BlockSpec((B, N), lambda r: (0, 0)),
        pl.BlockSpec((B, N), lambda r: (0, 0)),
    )
    max0, idx0, max1, idx1 = pl.pallas_call(
        _stage1_body,
        grid=(NR,),
        in_specs=in_specs,
        out_specs=out_specs,
        out_shape=out_shapes,
        compiler_params=pltpu.CompilerParams(
            dimension_semantics=("arbitrary",),
        ),
    )(scores_t)
    return max0, idx0, max1, idx1


_SC_INFO = None


def _sc_info():
    global _SC_INFO
    if _SC_INFO is None:
        info = plsc.get_sparse_core_info()
        _SC_INFO = (info.num_cores, info.num_subcores, info.num_lanes)
    return _SC_INFO


def _stage2(idx0, idx1, max0, max1):
    nc, ns, nl = _sc_info()
    nw = nc * ns
    chunks_per_b = nw // B
    chunk = N // chunks_per_b

    mesh = plsc.VectorSubcoreMesh(core_axis_name="c", subcore_axis_name="s")
    out_type = (
        jax.ShapeDtypeStruct((B, N), jnp.int32),
        jax.ShapeDtypeStruct((B, N), jnp.int32),
        jax.ShapeDtypeStruct((B, N), jnp.float32),
        jax.ShapeDtypeStruct((B, N), jnp.float32),
    )

    @functools.partial(
        pl.kernel,
        mesh=mesh,
        out_type=out_type,
        compiler_params=pltpu.CompilerParams(needs_layout_passes=False),
        scratch_types=[
            pltpu.VMEM((N,), jnp.int32),
            pltpu.VMEM((N,), jnp.int32),
            pltpu.VMEM((chunk,), jnp.float32),
            pltpu.VMEM((chunk,), jnp.float32),
            pltpu.VMEM((chunk,), jnp.int32),
            pltpu.VMEM((chunk,), jnp.int32),
            pltpu.VMEM((chunk,), jnp.float32),
            pltpu.VMEM((chunk,), jnp.float32),
            pltpu.SemaphoreType.DMA,
            pltpu.SemaphoreType.DMA,
            pltpu.SemaphoreType.DMA,
            pltpu.SemaphoreType.DMA,
        ],
    )
    def sc_kernel(idx0_hbm, idx1_hbm, max0_hbm, max1_hbm,
                  oi0_hbm, oi1_hbm, os0_hbm, os1_hbm,
                  idx0_v, idx1_v, max0_v, max1_v,
                  oi0_v, oi1_v, os0_v, os1_v,
                  sem0, sem1, sem2, sem3):
        wid = lax.axis_index("s") * nc + lax.axis_index("c")
        b = wid // chunks_per_b
        base = (wid % chunks_per_b) * chunk

        c0 = pltpu.async_copy(idx0_hbm.at[b], idx0_v, sem0)
        c1 = pltpu.async_copy(idx1_hbm.at[b], idx1_v, sem1)
        c2 = pltpu.async_copy(max0_hbm.at[b, pl.ds(base, chunk)], max0_v, sem2)
        c3 = pltpu.async_copy(max1_hbm.at[b, pl.ds(base, chunk)], max1_v, sem3)
        c0.wait()
        c1.wait()
        c2.wait()
        c3.wait()

        thr = jnp.float32(LOG2_T)
        for v in range(chunk // nl):
            s = v * nl
            ivec = lax.iota(jnp.int32, nl) + (base + s)
            a0 = idx0_v[pl.ds(base + s, nl)]
            g0 = plsc.load_gather(idx1_v, [a0])
            m0 = max0_v[pl.ds(s, nl)]
            valid0 = (g0 == ivec) & (m0 > thr)
            oi0_v[pl.ds(s, nl)] = jnp.where(valid0, a0, -1)
            os0_v[pl.ds(s, nl)] = jnp.where(valid0, m0, jnp.float32(0.0))
            a1 = idx1_v[pl.ds(base + s, nl)]
            g1 = plsc.load_gather(idx0_v, [a1])
            m1 = max1_v[pl.ds(s, nl)]
            valid1 = (g1 == ivec) & (m1 > thr)
            oi1_v[pl.ds(s, nl)] = jnp.where(valid1, a1, -1)
            os1_v[pl.ds(s, nl)] = jnp.where(valid1, m1, jnp.float32(0.0))

        o0 = pltpu.async_copy(oi0_v, oi0_hbm.at[b, pl.ds(base, chunk)], sem0)
        o1 = pltpu.async_copy(oi1_v, oi1_hbm.at[b, pl.ds(base, chunk)], sem1)
        o2 = pltpu.async_copy(os0_v, os0_hbm.at[b, pl.ds(base, chunk)], sem2)
        o3 = pltpu.async_copy(os1_v, os1_hbm.at[b, pl.ds(base, chunk)], sem3)
        o0.wait()
        o1.wait()
        o2.wait()
        o3.wait()

    return sc_kernel(idx0, idx1, max0, max1)


def kernel(scores):
    max0, idx0, max1, idx1 = _stage1(scores)
    return _stage2(idx0, idx1, max0, max1)
